# baseline (speedup 1.0000x reference)
import functools

import jax
import jax.numpy as jnp
from jax import lax
from jax.experimental import pallas as pl
from jax.experimental.pallas import tpu as pltpu
from jax.experimental.pallas import tpu_sc as plsc

LANES = 16
CHUNK = 128
IDX_PHASES = 2
N_SC = 2
N_TILES = 16
HALF = 128


def _fill_const_2d(ref, rows, cols, val):
    per_row = cols // LANES

    def body(k, _):
        r = k // per_row
        col0 = (k % per_row) * LANES
        ref[r, pl.ds(col0, LANES)] = jnp.full((LANES,), val, jnp.float32)
        return 0

    lax.fori_loop(0, rows * per_row, body, 0)


def _deg_body(n_nodes, n_pad, chunks_per_tile,
              dst_hbm, deg_out0, deg_out1, idx_v, ones_v, zeros_v, acc_sh, ssem):
    c = lax.axis_index("c")
    s = lax.axis_index("s")
    t = c * N_TILES + s
    pltpu.sync_copy(dst_hbm.at[pl.ds(t * chunks_per_tile, chunks_per_tile)], idx_v)
    _fill_const_2d(ones_v, chunks_per_tile, CHUNK, 1.0)
    for k in range(CHUNK // LANES):
        zeros_v[pl.ds(k * LANES, LANES)] = jnp.zeros((LANES,), jnp.float32)
    zrows = n_pad // N_TILES
    for z0 in range(0, zrows, CHUNK):
        zlen = min(CHUNK, zrows - z0)
        pltpu.sync_copy(zeros_v.at[pl.ds(0, zlen)],
                        acc_sh.at[pl.ds(s * zrows + z0, zlen)])
    plsc.subcore_barrier()

    wave = 10
    for w0 in range(0, chunks_per_tile, wave):
        wlen = min(wave, chunks_per_tile - w0)
        for j in range(w0, w0 + wlen):
            pltpu.async_copy(ones_v.at[j], acc_sh.at[idx_v.at[j]], ssem, add=True)
        for j in range(w0, w0 + wlen):
            pltpu.make_async_copy(ones_v.at[j], acc_sh.at[idx_v.at[j]], ssem).wait()
    plsc.subcore_barrier()

    @pl.when(c == 0)
    def _():
        pltpu.sync_copy(acc_sh.at[pl.ds(s * zrows, zrows)],
                        deg_out0.at[pl.ds(s * zrows, zrows)])

    @pl.when(c == 1)
    def _():
        pltpu.sync_copy(acc_sh.at[pl.ds(s * zrows, zrows)],
                        deg_out1.at[pl.ds(s * zrows, zrows)])


def _agg_body(n_nodes, n_pad, chunks_per_tile,
              srcs_hbm, dsts_hbm, y_hbm, agg_out,
              sidx_v, didx_v, buf0, buf1, acc_sh,
              sem0, sem1, ssem0, ssem1):
    c = lax.axis_index("c")
    s = lax.axis_index("s")
    _fill_const_2d(buf0, CHUNK, HALF, 0.0)
    zrows = n_pad // N_TILES
    for z0 in range(0, zrows, CHUNK):
        zlen = min(CHUNK, zrows - z0)
        pltpu.sync_copy(buf0.at[pl.ds(0, zlen)],
                        acc_sh.at[pl.ds(s * zrows + z0, zlen)])
    plsc.subcore_barrier()

    phase_chunks = chunks_per_tile // IDX_PHASES
    n_outer = phase_chunks // 2

    def gather(j, buf, sem):
        return pltpu.make_async_copy(y_hbm.at[sidx_v.at[j]], buf, sem)

    def scatter_start(j, buf, sem):
        pltpu.async_copy(buf, acc_sh.at[didx_v.at[j]], sem, add=True)

    def scatter_wait(j, buf, sem):
        pltpu.make_async_copy(buf, acc_sh.at[didx_v.at[j]], sem).wait()

    for phase in range(IDX_PHASES):
        base = s * chunks_per_tile + phase * phase_chunks
        pltpu.sync_copy(srcs_hbm.at[c, pl.ds(base, phase_chunks)], sidx_v)
        pltpu.sync_copy(dsts_hbm.at[pl.ds(base, phase_chunks)], didx_v)
        gather(0, buf0, sem0).start()
        gather(1, buf1, sem1).start()

        def outer(g, _):
            for bi, (buf, gsem, ssem) in enumerate(((buf0, sem0, ssem0),
                                                    (buf1, sem1, ssem1))):
                j = 2 * g + bi
                gather(j, buf, gsem).wait()
                scatter_start(j, buf, ssem)
                scatter_wait(j, buf, ssem)

                @pl.when(g < n_outer - 1)
                def _():
                    gather(j + 2, buf, gsem).start()
            return 0

        lax.fori_loop(0, n_outer, outer, 0)
    plsc.subcore_barrier()
    pltpu.sync_copy(acc_sh.at[pl.ds(s * zrows, zrows)],
                    agg_out.at[c, pl.ds(s * zrows, zrows)])


def _mm_body(x_ref, w_ref, d0_ref, d1_ref, y_ref):
    xw = jnp.dot(x_ref[...], w_ref[...], preferred_element_type=jnp.float32)
    dsum = d0_ref[...] + d1_ref[...] + 1.0
    dinv = lax.rsqrt(dsum)
    y = xw * dinv
    y_ref[0] = y[:, :HALF]
    y_ref[1] = y[:, HALF:]


def _out_body(agg_ref, y_ref, d0_ref, d1_ref, b_ref, o_ref):
    dsum = d0_ref[...] + d1_ref[...] + 1.0
    dinv = lax.rsqrt(dsum)
    lo = dinv * (agg_ref[0] + y_ref[0]) + b_ref[0, :HALF][None, :]
    hi = dinv * (agg_ref[1] + y_ref[1]) + b_ref[0, HALF:][None, :]
    o_ref[...] = jnp.concatenate([lo, hi], axis=1)


def kernel(x, edge_index, W, b):
    n, in_ch = x.shape
    out_ch = W.shape[1]
    e = edge_index.shape[1]

    pad_unit = N_TILES * 16
    n_pad = ((n + pad_unit - 1) // pad_unit) * pad_unit
    if n_pad == n:
        n_pad = n + pad_unit
    e_align = N_SC * N_TILES * CHUNK
    e_pad = ((e + e_align - 1) // e_align) * e_align
    if e_pad == e:
        e_pad = e + e_align
    n_fill = e_pad - e

    src = edge_index[0]
    dst = edge_index[1]
    fill = jnp.arange(n_fill, dtype=jnp.int32)
    src_p = jnp.concatenate([src, fill % n])
    dst_p = jnp.concatenate([dst, n + fill % (n_pad - n)])
    srcs2 = jnp.stack([src_p, src_p + n]).reshape(N_SC, e_pad // CHUNK, CHUNK)
    dsts = dst_p.reshape(e_pad // CHUNK, CHUNK)

    mesh = plsc.VectorSubcoreMesh(core_axis_name="c", subcore_axis_name="s")

    deg_chunks = e_pad // (N_SC * N_TILES * CHUNK)
    deg_call = pl.kernel(
        functools.partial(_deg_body, n, n_pad, deg_chunks),
        out_type=[jax.ShapeDtypeStruct((n_pad,), jnp.float32),
                  jax.ShapeDtypeStruct((n_pad,), jnp.float32)],
        scratch_types=[
            pltpu.VMEM((deg_chunks, CHUNK), jnp.int32),
            pltpu.VMEM((deg_chunks, CHUNK), jnp.float32),
            pltpu.VMEM((CHUNK,), jnp.float32),
            pltpu.VMEM_SHARED((n_pad,), jnp.float32),
            pltpu.SemaphoreType.DMA,
        ],
        mesh=mesh,
    )
    dp0, dp1 = deg_call(dsts)
    d0 = dp0.reshape(n_pad, 1)
    d1 = dp1.reshape(n_pad, 1)

    blk = 5000
    grid = n // blk
    y2 = pl.pallas_call(
        _mm_body,
        grid=(grid,),
        in_specs=[
            pl.BlockSpec((blk, in_ch), lambda i: (i, 0)),
            pl.BlockSpec((in_ch, out_ch), lambda i: (0, 0)),
            pl.BlockSpec((blk, 1), lambda i: (i, 0)),
            pl.BlockSpec((blk, 1), lambda i: (i, 0)),
        ],
        out_specs=pl.BlockSpec((N_SC, blk, HALF), lambda i: (0, i, 0)),
        out_shape=jax.ShapeDtypeStruct((N_SC, n, HALF), jnp.float32),
    )(x, W, d0, d1)

    chunks_per_tile = e_pad // (N_TILES * CHUNK)
    agg_call = pl.kernel(
        functools.partial(_agg_body, n, n_pad, chunks_per_tile),
        out_type=jax.ShapeDtypeStruct((N_SC, n_pad, HALF), jnp.float32),
        scratch_types=[
            pltpu.VMEM((chunks_per_tile // IDX_PHASES, CHUNK), jnp.int32),
            pltpu.VMEM((chunks_per_tile // IDX_PHASES, CHUNK), jnp.int32),
            pltpu.VMEM((CHUNK, HALF), jnp.float32),
            pltpu.VMEM((CHUNK, HALF), jnp.float32),
            pltpu.VMEM_SHARED((n_pad, HALF), jnp.float32),
            pltpu.SemaphoreType.DMA,
            pltpu.SemaphoreType.DMA,
            pltpu.SemaphoreType.DMA,
            pltpu.SemaphoreType.DMA,
        ],
        mesh=mesh,
    )
    agg2 = agg_call(srcs2, dsts, y2.reshape(N_SC * n, HALF))

    out = pl.pallas_call(
        _out_body,
        grid=(grid,),
        in_specs=[
            pl.BlockSpec((N_SC, blk, HALF), lambda i: (0, i, 0)),
            pl.BlockSpec((N_SC, blk, HALF), lambda i: (0, i, 0)),
            pl.BlockSpec((blk, 1), lambda i: (i, 0)),
            pl.BlockSpec((blk, 1), lambda i: (i, 0)),
            pl.BlockSpec((1, out_ch), lambda i: (0, 0)),
        ],
        out_specs=pl.BlockSpec((blk, out_ch), lambda i: (i, 0)),
        out_shape=jax.ShapeDtypeStruct((n, out_ch), jnp.float32),
    )(agg2, y2, d0, d1, b.reshape(1, out_ch))
    return out


# speedup vs baseline: 1.0025x; 1.0025x over previous
import functools

import jax
import jax.numpy as jnp
from jax import lax
from jax.experimental import pallas as pl
from jax.experimental.pallas import tpu as pltpu
from jax.experimental.pallas import tpu_sc as plsc

LANES = 16
CHUNK = 128
IDX_PHASES = 2
N_SC = 2
N_TILES = 16
HALF = 128


def _fill_const_2d(ref, rows, cols, val):
    per_row = cols // LANES

    def body(k, _):
        r = k // per_row
        col0 = (k % per_row) * LANES
        ref[r, pl.ds(col0, LANES)] = jnp.full((LANES,), val, jnp.float32)
        return 0

    lax.fori_loop(0, rows * per_row, body, 0)


def _deg_body(n_nodes, n_pad, chunks_per_tile,
              dst_hbm, deg_out0, deg_out1, idx_v, ones_v, zeros_v, acc_sh, ssem):
    c = lax.axis_index("c")
    s = lax.axis_index("s")
    t = c * N_TILES + s
    pltpu.sync_copy(dst_hbm.at[pl.ds(t * chunks_per_tile, chunks_per_tile)], idx_v)
    _fill_const_2d(ones_v, chunks_per_tile, CHUNK, 1.0)
    for k in range(CHUNK // LANES):
        zeros_v[pl.ds(k * LANES, LANES)] = jnp.zeros((LANES,), jnp.float32)
    zrows = n_pad // N_TILES
    for z0 in range(0, zrows, CHUNK):
        zlen = min(CHUNK, zrows - z0)
        pltpu.sync_copy(zeros_v.at[pl.ds(0, zlen)],
                        acc_sh.at[pl.ds(s * zrows + z0, zlen)])
    plsc.subcore_barrier()

    wave = 10
    for w0 in range(0, chunks_per_tile, wave):
        wlen = min(wave, chunks_per_tile - w0)
        for j in range(w0, w0 + wlen):
            pltpu.async_copy(ones_v.at[j], acc_sh.at[idx_v.at[j]], ssem, add=True)
        for j in range(w0, w0 + wlen):
            pltpu.make_async_copy(ones_v.at[j], acc_sh.at[idx_v.at[j]], ssem).wait()
    plsc.subcore_barrier()

    @pl.when(c == 0)
    def _():
        pltpu.sync_copy(acc_sh.at[pl.ds(s * zrows, zrows)],
                        deg_out0.at[pl.ds(s * zrows, zrows)])

    @pl.when(c == 1)
    def _():
        pltpu.sync_copy(acc_sh.at[pl.ds(s * zrows, zrows)],
                        deg_out1.at[pl.ds(s * zrows, zrows)])


def _agg_body(n_nodes, n_pad, chunks_per_tile,
              srcs_hbm, dsts_hbm, y_hbm, agg_out,
              sidx_v, didx_v, buf0, buf1, acc_sh,
              sem0, sem1, ssem0, ssem1):
    c = lax.axis_index("c")
    s = lax.axis_index("s")
    _fill_const_2d(buf0, CHUNK, HALF, 0.0)
    zrows = n_pad // N_TILES
    for z0 in range(0, zrows, CHUNK):
        zlen = min(CHUNK, zrows - z0)
        pltpu.sync_copy(buf0.at[pl.ds(0, zlen)],
                        acc_sh.at[pl.ds(s * zrows + z0, zlen)])
    plsc.subcore_barrier()

    phase_chunks = chunks_per_tile // IDX_PHASES
    n_outer = phase_chunks // 2

    def gather(j, buf, sem):
        return pltpu.make_async_copy(y_hbm.at[sidx_v.at[j]], buf, sem)

    def scatter_start(j, buf, sem):
        pltpu.async_copy(buf, acc_sh.at[didx_v.at[j]], sem, add=True)

    def scatter_wait(j, buf, sem):
        pltpu.make_async_copy(buf, acc_sh.at[didx_v.at[j]], sem).wait()

    for phase in range(IDX_PHASES):
        base = s * chunks_per_tile + phase * phase_chunks
        pltpu.sync_copy(srcs_hbm.at[c, pl.ds(base, phase_chunks)], sidx_v)
        pltpu.sync_copy(dsts_hbm.at[pl.ds(base, phase_chunks)], didx_v)
        gather(0, buf0, sem0).start()
        gather(1, buf1, sem1).start()

        def outer(g, _):
            for bi, (buf, gsem, ssem) in enumerate(((buf0, sem0, ssem0),
                                                    (buf1, sem1, ssem1))):
                j = 2 * g + bi
                gather(j, buf, gsem).wait()
                scatter_start(j, buf, ssem)
                scatter_wait(j, buf, ssem)

                @pl.when(g < n_outer - 1)
                def _():
                    gather(j + 2, buf, gsem).start()
            return 0

        lax.fori_loop(0, n_outer, outer, 0)
    plsc.subcore_barrier()
    pltpu.sync_copy(acc_sh.at[pl.ds(s * zrows, zrows)],
                    agg_out.at[c, pl.ds(s * zrows, zrows)])


def _mm_body(x_ref, w_ref, d0_ref, d1_ref, y_ref):
    xw = jnp.dot(x_ref[...], w_ref[...], preferred_element_type=jnp.float32)
    dsum = d0_ref[...] + d1_ref[...] + 1.0
    dinv = lax.rsqrt(dsum)
    y = xw * dinv
    y_ref[0] = y[:, :HALF]
    y_ref[1] = y[:, HALF:]


def _out_body(agg_ref, y_ref, d0_ref, d1_ref, b_ref, o_ref):
    dsum = d0_ref[...] + d1_ref[...] + 1.0
    dinv = lax.rsqrt(dsum)
    lo = dinv * (agg_ref[0] + y_ref[0]) + b_ref[0, :HALF][None, :]
    hi = dinv * (agg_ref[1] + y_ref[1]) + b_ref[0, HALF:][None, :]
    o_ref[...] = jnp.concatenate([lo, hi], axis=1)


def kernel(x, edge_index, W, b):
    n, in_ch = x.shape
    out_ch = W.shape[1]
    e = edge_index.shape[1]

    pad_unit = N_TILES * 16
    n_pad = ((n + pad_unit - 1) // pad_unit) * pad_unit
    if n_pad == n:
        n_pad = n + pad_unit
    e_align = N_SC * N_TILES * CHUNK
    e_pad = ((e + e_align - 1) // e_align) * e_align
    if e_pad == e:
        e_pad = e + e_align
    n_fill = e_pad - e

    src = edge_index[0]
    dst = edge_index[1]
    fill = jnp.arange(n_fill, dtype=jnp.int32)
    src_p = jnp.concatenate([src, fill % n])
    dst_p = jnp.concatenate([dst, n + fill % (n_pad - n)])
    srcs2 = jnp.stack([src_p, src_p + n]).reshape(N_SC, e_pad // CHUNK, CHUNK)
    dsts = dst_p.reshape(e_pad // CHUNK, CHUNK)

    mesh = plsc.VectorSubcoreMesh(core_axis_name="c", subcore_axis_name="s")

    deg_chunks = e_pad // (N_SC * N_TILES * CHUNK)
    deg_call = pl.kernel(
        functools.partial(_deg_body, n, n_pad, deg_chunks),
        out_type=[jax.ShapeDtypeStruct((n_pad,), jnp.float32),
                  jax.ShapeDtypeStruct((n_pad,), jnp.float32)],
        scratch_types=[
            pltpu.VMEM((deg_chunks, CHUNK), jnp.int32),
            pltpu.VMEM((deg_chunks, CHUNK), jnp.float32),
            pltpu.VMEM((CHUNK,), jnp.float32),
            pltpu.VMEM_SHARED((n_pad,), jnp.float32),
            pltpu.SemaphoreType.DMA,
        ],
        mesh=mesh,
    )
    dp0, dp1 = deg_call(dsts)
    d0 = dp0.reshape(n_pad, 1)
    d1 = dp1.reshape(n_pad, 1)

    blk = 2000
    grid = n // blk
    y2 = pl.pallas_call(
        _mm_body,
        grid=(grid,),
        in_specs=[
            pl.BlockSpec((blk, in_ch), lambda i: (i, 0)),
            pl.BlockSpec((in_ch, out_ch), lambda i: (0, 0)),
            pl.BlockSpec((blk, 1), lambda i: (i, 0)),
            pl.BlockSpec((blk, 1), lambda i: (i, 0)),
        ],
        out_specs=pl.BlockSpec((N_SC, blk, HALF), lambda i: (0, i, 0)),
        out_shape=jax.ShapeDtypeStruct((N_SC, n, HALF), jnp.float32),
    )(x, W, d0, d1)

    chunks_per_tile = e_pad // (N_TILES * CHUNK)
    agg_call = pl.kernel(
        functools.partial(_agg_body, n, n_pad, chunks_per_tile),
        out_type=jax.ShapeDtypeStruct((N_SC, n_pad, HALF), jnp.float32),
        scratch_types=[
            pltpu.VMEM((chunks_per_tile // IDX_PHASES, CHUNK), jnp.int32),
            pltpu.VMEM((chunks_per_tile // IDX_PHASES, CHUNK), jnp.int32),
            pltpu.VMEM((CHUNK, HALF), jnp.float32),
            pltpu.VMEM((CHUNK, HALF), jnp.float32),
            pltpu.VMEM_SHARED((n_pad, HALF), jnp.float32),
            pltpu.SemaphoreType.DMA,
            pltpu.SemaphoreType.DMA,
            pltpu.SemaphoreType.DMA,
            pltpu.SemaphoreType.DMA,
        ],
        mesh=mesh,
    )
    agg2 = agg_call(srcs2, dsts, y2.reshape(N_SC * n, HALF))

    out = pl.pallas_call(
        _out_body,
        grid=(grid,),
        in_specs=[
            pl.BlockSpec((N_SC, blk, HALF), lambda i: (0, i, 0)),
            pl.BlockSpec((N_SC, blk, HALF), lambda i: (0, i, 0)),
            pl.BlockSpec((blk, 1), lambda i: (i, 0)),
            pl.BlockSpec((blk, 1), lambda i: (i, 0)),
            pl.BlockSpec((1, out_ch), lambda i: (0, 0)),
        ],
        out_specs=pl.BlockSpec((blk, out_ch), lambda i: (i, 0)),
        out_shape=jax.ShapeDtypeStruct((n, out_ch), jnp.float32),
    )(agg2, y2, d0, d1, b.reshape(1, out_ch))
    return out
